# baseline (speedup 1.0000x reference)
import functools

import jax
import jax.numpy as jnp
from jax import lax
from jax.experimental import pallas as pl
from jax.experimental.pallas import tpu as pltpu
from jax.experimental.pallas import tpu_sc as plsc

_B, _S, _D = 2, 2048, 768
_N_IN, _N_PROC, _D_R = 4096, 2048, 256
_K_IN, _K_PROC = 2048, 1024
_TEMP = 0.5


def _gelu(v):
    return 0.5 * v * (1.0 + lax.erf(v * (2.0 ** -0.5)))


def _kth_threshold(vals, k, iters=48):
    lo = jnp.min(vals, axis=-1, keepdims=True)
    hi = jnp.max(vals, axis=-1, keepdims=True) + 1.0

    def step(_, lh):
        lo_, hi_ = lh
        mid = 0.5 * (lo_ + hi_)
        cnt = jnp.sum((vals >= mid).astype(jnp.float32), axis=-1,
                      keepdims=True)
        pred = cnt >= k
        return jnp.where(pred, mid, lo_), jnp.where(pred, hi_, mid)

    lo, hi = lax.fori_loop(0, iters, step, (lo, hi))
    return lo


def _pack_pairs(tab):
    h = tab.shape[1] // 2
    b = tab.astype(jnp.bfloat16)
    lo = lax.bitcast_convert_type(b[:, :h], jnp.uint16).astype(jnp.uint32)
    hi = lax.bitcast_convert_type(b[:, h:], jnp.uint16).astype(jnp.uint32)
    return lax.bitcast_convert_type(lo | (hi << 16), jnp.int32)


def _router_body(x_ref, w1_ref, b1_ref, g_ref, bln_ref, w2_ref, b2_ref,
                 nk_ref, logits_ref, thr_ref):
    gc = jnp.max(x_ref[...], axis=1)
    h = lax.dot_general(gc, w1_ref[...], (((1,), (1,)), ((), ())),
                        preferred_element_type=jnp.float32) + b1_ref[...]
    h = _gelu(h)
    mu = jnp.mean(h, axis=-1, keepdims=True)
    var = jnp.mean((h - mu) ** 2, axis=-1, keepdims=True)
    h = (h - mu) / jnp.sqrt(var + 1e-5) * g_ref[...] + bln_ref[...]
    q = lax.dot_general(h, w2_ref[...], (((1,), (1,)), ((), ())),
                        preferred_element_type=jnp.float32) + b2_ref[...]
    logits = lax.dot_general(
        q, nk_ref[...], (((1,), (1,)), ((), ())),
        preferred_element_type=jnp.float32) * (_D_R ** -0.5)
    logits_ref[...] = logits
    thr_ref[...] = jnp.broadcast_to(_kth_threshold(logits, _K_IN), (_B, 128))


def _router(x, w1, b1, g, bln, w2, b2, nk):
    return pl.pallas_call(
        _router_body,
        out_shape=[
            jax.ShapeDtypeStruct((_B, _N_IN), jnp.float32),
            jax.ShapeDtypeStruct((_B, 128), jnp.float32),
        ],
    )(x, w1, b1, g, bln, w2, b2, nk)


def _sc_gather2(ip_tab, pwt_tab, idx_flat, rows_total):
    info = plsc.get_sparse_core_info()
    nc, ns = info.num_cores, info.num_subcores
    per_w = rows_total // (nc * ns)
    chunk = 64
    d1, d2 = _D, _N_PROC // 2
    mesh = plsc.VectorSubcoreMesh(core_axis_name="c", subcore_axis_name="s")

    @functools.partial(
        pl.kernel, mesh=mesh,
        out_type=[
            jax.ShapeDtypeStruct((rows_total, d1), jnp.float32),
            jax.ShapeDtypeStruct((rows_total, d2), jnp.int32),
        ],
        scratch_types=[
            pltpu.VMEM((per_w,), jnp.int32),
            pltpu.VMEM((chunk, d1), jnp.float32),
            pltpu.VMEM((chunk, d2), jnp.int32),
            pltpu.SemaphoreType.DMA,
            pltpu.SemaphoreType.DMA,
        ],
    )
    def k(ip_hbm, pwt_hbm, idx_hbm, out1_hbm, out2_hbm,
          idx_v, buf1, buf2, sem1, sem2):
        wid = lax.axis_index("s") * nc + lax.axis_index("c")
        base = wid * per_w
        pltpu.sync_copy(idx_hbm.at[pl.ds(base, per_w)], idx_v)
        for c in range(per_w // chunk):
            off = base + c * chunk
            ids = idx_v.at[pl.ds(c * chunk, chunk)]
            cp1 = pltpu.async_copy(ip_hbm.at[ids], buf1, sem1)
            cp2 = pltpu.async_copy(pwt_hbm.at[ids], buf2, sem2)
            cp1.wait()
            pltpu.sync_copy(buf1, out1_hbm.at[pl.ds(off, chunk)])
            cp2.wait()
            pltpu.sync_copy(buf2, out2_hbm.at[pl.ds(off, chunk)])

    return k(ip_tab, pwt_tab, idx_flat)


def _selin_body(x_ref, ip_ref, out_ref):
    out_ref[0] = _gelu(lax.dot_general(
        x_ref[0], ip_ref[0].astype(jnp.bfloat16), (((1,), (1,)), ((), ())),
        preferred_element_type=jnp.float32)).astype(jnp.bfloat16)


def _selin(x_bf, ipsel):
    bs = 512
    return pl.pallas_call(
        _selin_body,
        grid=(_B, _S // bs),
        in_specs=[
            pl.BlockSpec((1, bs, _D), lambda b, s: (b, s, 0)),
            pl.BlockSpec((1, _K_IN, _D), lambda b, s: (b, 0, 0)),
        ],
        out_specs=pl.BlockSpec((1, bs, _K_IN), lambda b, s: (b, s, 0)),
        out_shape=jax.ShapeDtypeStruct((_B, _S, _K_IN), jnp.bfloat16),
    )(x_bf, ipsel)


def _pa_body(sel_ref, pwt_ref, pa_ref, ssum_ref, wlo_ref, whi_ref):
    s = pl.program_id(1)
    half = _N_PROC // 2

    @pl.when(s == 0)
    def _():
        w32 = pwt_ref[0]
        wlo_ref[...] = lax.bitcast_convert_type(
            w32 << 16, jnp.float32).astype(jnp.bfloat16)
        whi_ref[...] = lax.bitcast_convert_type(
            w32 & jnp.int32(-65536), jnp.float32).astype(jnp.bfloat16)

    sel = sel_ref[0]
    dims = (((1,), (0,)), ((), ()))
    a_lo = _gelu(lax.dot_general(sel, wlo_ref[...], dims,
                                 preferred_element_type=jnp.float32))
    a_hi = _gelu(lax.dot_general(sel, whi_ref[...], dims,
                                 preferred_element_type=jnp.float32))
    pa_ref[0, :, :half] = a_lo.astype(jnp.bfloat16)
    pa_ref[0, :, half:] = a_hi.astype(jnp.bfloat16)
    cs_lo = jnp.sum(a_lo, axis=0, keepdims=True)
    cs_hi = jnp.sum(a_hi, axis=0, keepdims=True)

    @pl.when(s == 0)
    def _():
        ssum_ref[0, :, :half] = cs_lo
        ssum_ref[0, :, half:] = cs_hi

    @pl.when(s > 0)
    def _():
        ssum_ref[0, :, :half] = ssum_ref[0, :, :half] + cs_lo
        ssum_ref[0, :, half:] = ssum_ref[0, :, half:] + cs_hi


def _process_acts(sel_in, pwtsel):
    bs = 512
    return pl.pallas_call(
        _pa_body,
        grid=(_B, _S // bs),
        in_specs=[
            pl.BlockSpec((1, bs, _K_IN), lambda b, s: (b, s, 0)),
            pl.BlockSpec((1, _K_IN, _N_PROC // 2), lambda b, s: (b, 0, 0)),
        ],
        out_specs=[
            pl.BlockSpec((1, bs, _N_PROC), lambda b, s: (b, s, 0)),
            pl.BlockSpec((1, 1, _N_PROC), lambda b, s: (b, 0, 0)),
        ],
        out_shape=[
            jax.ShapeDtypeStruct((_B, _S, _N_PROC), jnp.bfloat16),
            jax.ShapeDtypeStruct((_B, 1, _N_PROC), jnp.float32),
        ],
        scratch_shapes=[
            pltpu.VMEM((_K_IN, _N_PROC // 2), jnp.bfloat16),
            pltpu.VMEM((_K_IN, _N_PROC // 2), jnp.bfloat16),
        ],
    )(sel_in, pwtsel)


def _out_body(pa_ref, po_ref, ssum_ref, out_ref, mask_ref):
    s = pl.program_id(1)

    @pl.when(s == 0)
    def _():
        sc = ssum_ref[0]
        thr = _kth_threshold(sc, _K_PROC)
        mask_ref[...] = (sc >= thr).astype(jnp.float32)

    po_m = (po_ref[...] * mask_ref[0][:, None]).astype(jnp.bfloat16)
    out_ref[0] = lax.dot_general(pa_ref[0], po_m, (((1,), (0,)), ((), ())),
                                 preferred_element_type=jnp.float32)


def _out_mm(pa, po, ssum):
    bs = 512
    return pl.pallas_call(
        _out_body,
        grid=(_B, _S // bs),
        in_specs=[
            pl.BlockSpec((1, bs, _N_PROC), lambda b, s: (b, s, 0)),
            pl.BlockSpec((_N_PROC, _D), lambda b, s: (0, 0)),
            pl.BlockSpec((1, 1, _N_PROC), lambda b, s: (b, 0, 0)),
        ],
        out_specs=pl.BlockSpec((1, bs, _D), lambda b, s: (b, s, 0)),
        out_shape=jax.ShapeDtypeStruct((_B, _S, _D), jnp.float32),
        scratch_shapes=[pltpu.VMEM((1, _N_PROC), jnp.float32)],
    )(pa, po, ssum)


def kernel(x, W1, b1, ln_g, ln_b, W2, b2, neuron_keys, input_patterns,
           process_weights, process_outputs, k_input, k_process):
    logits, thr = _router(x, W1, b1, ln_g, ln_b, W2, b2, neuron_keys)

    sel = logits >= thr[:, :1]
    seli = sel.astype(jnp.int32)
    pos = jnp.cumsum(seli, axis=1) - seli
    scat = jnp.where(sel, pos, _K_IN)
    ar = jnp.broadcast_to(jnp.arange(_N_IN, dtype=jnp.int32)[None],
                          (_B, _N_IN))
    idx1 = jnp.zeros((_B, _K_IN), jnp.int32).at[
        jnp.arange(_B)[:, None], scat].set(ar, mode="drop")
    idx1_flat = idx1.reshape(-1)

    pwt_p = _pack_pairs(process_weights.T)
    ipsel, pwtsel = _sc_gather2(input_patterns, pwt_p, idx1_flat,
                                _B * _K_IN)
    ipsel = ipsel.reshape(_B, _K_IN, _D)
    pwtsel = pwtsel.reshape(_B, _K_IN, _N_PROC // 2)

    sel_in = _selin(x.astype(jnp.bfloat16), ipsel)
    pa, ssum = _process_acts(sel_in, pwtsel)
    return _out_mm(pa, process_outputs, ssum)


# speedup vs baseline: 1.1559x; 1.1559x over previous
import functools

import jax
import jax.numpy as jnp
from jax import lax
from jax.experimental import pallas as pl
from jax.experimental.pallas import tpu as pltpu
from jax.experimental.pallas import tpu_sc as plsc

_B, _S, _D = 2, 2048, 768
_N_IN, _N_PROC, _D_R = 4096, 2048, 256
_K_IN, _K_PROC = 2048, 1024
_TEMP = 0.5


def _gelu(v):
    return 0.5 * v * (1.0 + lax.erf(v * (2.0 ** -0.5)))


def _kth_threshold(vals, k, iters=48):
    lo = jnp.min(vals, axis=-1, keepdims=True)
    hi = jnp.max(vals, axis=-1, keepdims=True) + 1.0

    def step(_, lh):
        lo_, hi_ = lh
        mid = 0.5 * (lo_ + hi_)
        cnt = jnp.sum((vals >= mid).astype(jnp.float32), axis=-1,
                      keepdims=True)
        pred = cnt >= k
        return jnp.where(pred, mid, lo_), jnp.where(pred, hi_, mid)

    lo, hi = lax.fori_loop(0, iters, step, (lo, hi))
    return lo


def _pack_pairs(tab):
    h = tab.shape[1] // 2
    b = tab.astype(jnp.bfloat16)
    lo = lax.bitcast_convert_type(b[:, :h], jnp.uint16).astype(jnp.uint32)
    hi = lax.bitcast_convert_type(b[:, h:], jnp.uint16).astype(jnp.uint32)
    return lax.bitcast_convert_type(lo | (hi << 16), jnp.int32)


def _router_body(x_ref, w1_ref, b1_ref, g_ref, bln_ref, w2_ref, b2_ref,
                 nk_ref, logits_ref):
    gc = jnp.max(x_ref[...], axis=1)
    h = lax.dot_general(gc, w1_ref[...], (((1,), (1,)), ((), ())),
                        preferred_element_type=jnp.float32) + b1_ref[...]
    h = _gelu(h)
    mu = jnp.mean(h, axis=-1, keepdims=True)
    var = jnp.mean((h - mu) ** 2, axis=-1, keepdims=True)
    h = (h - mu) / jnp.sqrt(var + 1e-5) * g_ref[...] + bln_ref[...]
    q = lax.dot_general(h, w2_ref[...], (((1,), (1,)), ((), ())),
                        preferred_element_type=jnp.float32) + b2_ref[...]
    logits_ref[...] = lax.dot_general(
        q, nk_ref[...], (((1,), (1,)), ((), ())),
        preferred_element_type=jnp.float32) * (_D_R ** -0.5)


def _router(x, w1, b1, g, bln, w2, b2, nk):
    return pl.pallas_call(
        _router_body,
        out_shape=jax.ShapeDtypeStruct((_B, _N_IN), jnp.float32),
    )(x, w1, b1, g, bln, w2, b2, nk)


def _sc_gather2(ip_tab, pwt_tab, idx_flat, rows_total):
    info = plsc.get_sparse_core_info()
    nc, ns = info.num_cores, info.num_subcores
    per_w = rows_total // (nc * ns)
    chunk = 64
    d1, d2 = _D, _N_PROC // 2
    mesh = plsc.VectorSubcoreMesh(core_axis_name="c", subcore_axis_name="s")

    @functools.partial(
        pl.kernel, mesh=mesh,
        out_type=[
            jax.ShapeDtypeStruct((rows_total, d1), jnp.float32),
            jax.ShapeDtypeStruct((rows_total, d2), jnp.int32),
        ],
        scratch_types=[
            pltpu.VMEM((per_w,), jnp.int32),
            pltpu.VMEM((chunk, d1), jnp.float32),
            pltpu.VMEM((chunk, d2), jnp.int32),
            pltpu.SemaphoreType.DMA,
            pltpu.SemaphoreType.DMA,
        ],
    )
    def k(ip_hbm, pwt_hbm, idx_hbm, out1_hbm, out2_hbm,
          idx_v, buf1, buf2, sem1, sem2):
        wid = lax.axis_index("s") * nc + lax.axis_index("c")
        base = wid * per_w
        pltpu.sync_copy(idx_hbm.at[pl.ds(base, per_w)], idx_v)
        for c in range(per_w // chunk):
            off = base + c * chunk
            ids = idx_v.at[pl.ds(c * chunk, chunk)]
            cp1 = pltpu.async_copy(ip_hbm.at[ids], buf1, sem1)
            cp2 = pltpu.async_copy(pwt_hbm.at[ids], buf2, sem2)
            cp1.wait()
            pltpu.sync_copy(buf1, out1_hbm.at[pl.ds(off, chunk)])
            cp2.wait()
            pltpu.sync_copy(buf2, out2_hbm.at[pl.ds(off, chunk)])

    return k(ip_tab, pwt_tab, idx_flat)


def _selin_body(x_ref, ip_ref, out_ref):
    out_ref[0] = _gelu(lax.dot_general(
        x_ref[0], ip_ref[0].astype(jnp.bfloat16), (((1,), (1,)), ((), ())),
        preferred_element_type=jnp.float32)).astype(jnp.bfloat16)


def _selin(x_bf, ipsel):
    bs = 512
    return pl.pallas_call(
        _selin_body,
        grid=(_B, _S // bs),
        in_specs=[
            pl.BlockSpec((1, bs, _D), lambda b, s: (b, s, 0)),
            pl.BlockSpec((1, _K_IN, _D), lambda b, s: (b, 0, 0)),
        ],
        out_specs=pl.BlockSpec((1, bs, _K_IN), lambda b, s: (b, s, 0)),
        out_shape=jax.ShapeDtypeStruct((_B, _S, _K_IN), jnp.bfloat16),
    )(x_bf, ipsel)


def _pa_body(sel_ref, pwt_ref, pa_ref, ssum_ref, wlo_ref, whi_ref):
    s = pl.program_id(1)
    half = _N_PROC // 2

    @pl.when(s == 0)
    def _():
        w32 = pwt_ref[0]
        wlo_ref[...] = lax.bitcast_convert_type(
            w32 << 16, jnp.float32).astype(jnp.bfloat16)
        whi_ref[...] = lax.bitcast_convert_type(
            w32 & jnp.int32(-65536), jnp.float32).astype(jnp.bfloat16)

    sel = sel_ref[0]
    dims = (((1,), (0,)), ((), ()))
    a_lo = _gelu(lax.dot_general(sel, wlo_ref[...], dims,
                                 preferred_element_type=jnp.float32))
    a_hi = _gelu(lax.dot_general(sel, whi_ref[...], dims,
                                 preferred_element_type=jnp.float32))
    pa_ref[0, :, :half] = a_lo.astype(jnp.bfloat16)
    pa_ref[0, :, half:] = a_hi.astype(jnp.bfloat16)
    cs_lo = jnp.sum(a_lo, axis=0, keepdims=True)
    cs_hi = jnp.sum(a_hi, axis=0, keepdims=True)

    @pl.when(s == 0)
    def _():
        ssum_ref[0, :, :half] = cs_lo
        ssum_ref[0, :, half:] = cs_hi

    @pl.when(s > 0)
    def _():
        ssum_ref[0, :, :half] = ssum_ref[0, :, :half] + cs_lo
        ssum_ref[0, :, half:] = ssum_ref[0, :, half:] + cs_hi


def _process_acts(sel_in, pwtsel):
    bs = 512
    return pl.pallas_call(
        _pa_body,
        grid=(_B, _S // bs),
        in_specs=[
            pl.BlockSpec((1, bs, _K_IN), lambda b, s: (b, s, 0)),
            pl.BlockSpec((1, _K_IN, _N_PROC // 2), lambda b, s: (b, 0, 0)),
        ],
        out_specs=[
            pl.BlockSpec((1, bs, _N_PROC), lambda b, s: (b, s, 0)),
            pl.BlockSpec((1, 1, _N_PROC), lambda b, s: (b, 0, 0)),
        ],
        out_shape=[
            jax.ShapeDtypeStruct((_B, _S, _N_PROC), jnp.bfloat16),
            jax.ShapeDtypeStruct((_B, 1, _N_PROC), jnp.float32),
        ],
        scratch_shapes=[
            pltpu.VMEM((_K_IN, _N_PROC // 2), jnp.bfloat16),
            pltpu.VMEM((_K_IN, _N_PROC // 2), jnp.bfloat16),
        ],
    )(sel_in, pwtsel)


def _out_body(pa_ref, po_ref, idx2_ref, out_ref, mask_ref):
    s = pl.program_id(1)

    @pl.when(s == 0)
    def _():
        ids = idx2_ref[0]
        piota = lax.broadcasted_iota(jnp.int32, (_K_PROC, _N_PROC), 1)
        hits = (ids[0][:, None] == piota).astype(jnp.float32)
        mask_ref[...] = jnp.max(hits, axis=0, keepdims=True)

    po_m = (po_ref[...] * mask_ref[0][:, None]).astype(jnp.bfloat16)
    out_ref[0] = lax.dot_general(pa_ref[0], po_m, (((1,), (0,)), ((), ())),
                                 preferred_element_type=jnp.float32)


def _out_mm(pa, po, idx2):
    bs = 512
    return pl.pallas_call(
        _out_body,
        grid=(_B, _S // bs),
        in_specs=[
            pl.BlockSpec((1, bs, _N_PROC), lambda b, s: (b, s, 0)),
            pl.BlockSpec((_N_PROC, _D), lambda b, s: (0, 0)),
            pl.BlockSpec((1, 1, _K_PROC), lambda b, s: (b, 0, 0)),
        ],
        out_specs=pl.BlockSpec((1, bs, _D), lambda b, s: (b, s, 0)),
        out_shape=jax.ShapeDtypeStruct((_B, _S, _D), jnp.float32),
        scratch_shapes=[pltpu.VMEM((1, _N_PROC), jnp.float32)],
    )(pa, po, idx2)


def kernel(x, W1, b1, ln_g, ln_b, W2, b2, neuron_keys, input_patterns,
           process_weights, process_outputs, k_input, k_process):
    logits = _router(x, W1, b1, ln_g, ln_b, W2, b2, neuron_keys)
    _, input_idx = lax.top_k(logits, _K_IN)
    idx1_flat = input_idx.reshape(-1).astype(jnp.int32)

    pwt_p = _pack_pairs(process_weights.T)
    ipsel, pwtsel = _sc_gather2(input_patterns, pwt_p, idx1_flat,
                                _B * _K_IN)
    ipsel = ipsel.reshape(_B, _K_IN, _D)
    pwtsel = pwtsel.reshape(_B, _K_IN, _N_PROC // 2)

    sel_in = _selin(x.astype(jnp.bfloat16), ipsel)
    pa, ssum = _process_acts(sel_in, pwtsel)
    _, process_idx = lax.top_k(ssum.reshape(_B, _N_PROC), _K_PROC)
    idx2 = process_idx.astype(jnp.int32).reshape(_B, 1, _K_PROC)
    return _out_mm(pa, process_outputs, idx2)
